# baseline (speedup 1.0000x reference)
import functools

import jax
import jax.numpy as jnp
from jax.experimental import pallas as pl

_IGNORE_INDEX = -100
_MAX_ITER = 3


def _argmax_body(x_ref, out_ref):
    x = x_ref[0]
    pred = jnp.argmax(x, axis=-1, keepdims=True)
    out_ref[0] = pred.astype(jnp.int32)


def _assign_body(depth_ref, pred_ref, lab_ref, mask_ref, valid_ref, full_ref,
                 la_out, full_out, *, S, CH):
    c = pl.program_id(1)
    pred = pred_ref[0]
    lab = lab_ref[0]
    maskv = mask_ref[0]
    valid = valid_ref[0]
    d = depth_ref[0, 0]

    s_row = jax.lax.broadcasted_iota(jnp.int32, (1, S), 1)
    cont = (pred != lab) & (s_row < (S - 1)) & (lab != _IGNORE_INDEX)
    la = jnp.where(cont, d + 1, d)
    la = jnp.minimum(la, _MAX_ITER)
    la = jnp.where(valid == 1, la, _IGNORE_INDEX).astype(jnp.int32)

    proposal = jnp.where(la == _IGNORE_INDEX, 0, la).astype(jnp.float32)
    mf = maskv.astype(jnp.float32)

    jj = jax.lax.broadcasted_iota(jnp.int32, (S, CH), 0)
    ss = jax.lax.broadcasted_iota(jnp.int32, (S, CH), 1) + c * CH
    upper = (jj <= ss).astype(jnp.float32)
    pos = jnp.dot(mf, upper, preferred_element_type=jnp.float32)
    pos_i = jnp.clip(pos.astype(jnp.int32) - 1, 0, S - 1)

    onehot_t = (jnp.broadcast_to(pos_i, (S, CH)) == jj).astype(jnp.float32)
    gathered = jnp.dot(proposal, onehot_t,
                       preferred_element_type=jnp.float32)
    g_i = gathered.astype(jnp.int32)

    chunk = pl.ds(c * CH, CH)
    pred_c = pred_ref[0, :, chunk]
    lab_c = lab_ref[0, :, chunk]
    valid_c = valid_ref[0, :, chunk]
    mask_c = mask_ref[0, :, chunk]
    full_c = full_ref[0, :, chunk]
    s_row_c = jax.lax.broadcasted_iota(jnp.int32, (1, CH), 1) + c * CH
    cont_c = (pred_c != lab_c) & (s_row_c < (S - 1)) & (lab_c != _IGNORE_INDEX)
    la_c = jnp.where(cont_c, d + 1, d)
    la_c = jnp.minimum(la_c, _MAX_ITER)
    la_c = jnp.where(valid_c == 1, la_c, _IGNORE_INDEX).astype(jnp.int32)

    tmp = jnp.where(mask_c != 0, g_i, 0)
    la_out[0] = la_c
    full_out[0] = jnp.maximum(full_c, tmp)


def kernel(active_logits, active_labels_shifted, iter_depth,
           current_iter_mask, active_valid_mask, full_labels):
    B, S, V = active_logits.shape
    BS = 256
    n_sblk = S // BS

    predicted = pl.pallas_call(
        _argmax_body,
        grid=(B, n_sblk),
        in_specs=[pl.BlockSpec((1, BS, V), lambda b, s: (b, s, 0))],
        out_specs=pl.BlockSpec((1, BS, 1), lambda b, s: (b * (S // BS) + s, 0, 0)),
        out_shape=jax.ShapeDtypeStruct((B * n_sblk, BS, 1), jnp.int32),
    )(active_logits)
    predicted = predicted.reshape(B, 1, S)

    CH = 512
    n_cblk = S // CH
    depth = jnp.asarray(iter_depth, dtype=jnp.int32).reshape(1, 1)
    lab = active_labels_shifted.astype(jnp.int32).reshape(B, 1, S)
    maskv = current_iter_mask.astype(jnp.int32).reshape(B, 1, S)
    valid = active_valid_mask.astype(jnp.int32).reshape(B, 1, S)
    full = full_labels.reshape(B, 1, S)

    row_spec = pl.BlockSpec((1, 1, S), lambda b, c: (b, 0, 0))
    out_spec = pl.BlockSpec((1, 1, CH), lambda b, c: (b * (S // CH) + c, 0, 0))
    la, full_new = pl.pallas_call(
        functools.partial(_assign_body, S=S, CH=CH),
        grid=(B, n_cblk),
        in_specs=[
            pl.BlockSpec((1, 1), lambda b, c: (0, 0)),
            row_spec, row_spec, row_spec, row_spec, row_spec,
        ],
        out_specs=[out_spec, out_spec],
        out_shape=[
            jax.ShapeDtypeStruct((B * n_cblk, 1, CH), jnp.int32),
            jax.ShapeDtypeStruct((B * n_cblk, 1, CH), jnp.int32),
        ],
    )(depth, predicted, lab, maskv, valid, full)

    return la.reshape(B, S), full_new.reshape(B, S)


# speedup vs baseline: 1.5587x; 1.5587x over previous
import functools

import jax
import jax.numpy as jnp
from jax import lax
from jax.experimental import pallas as pl
from jax.experimental.pallas import tpu as pltpu
from jax.experimental.pallas import tpu_sc as plsc

_IGNORE_INDEX = -100
_MAX_ITER = 3
_LANES = 16
_NUM_CORES = 2
_NUM_SUBCORES = 16


def _argmax_body(x_ref, out_ref):
    x = x_ref[0]
    pred = jnp.argmax(x, axis=-1, keepdims=True)
    out_ref[0] = pred.astype(jnp.int32)


def _sc_assign_body(pred_hbm, lab_hbm, valid_hbm, mask_hbm, full_hbm,
                    depth_hbm, la_hbm, full_out_hbm,
                    pred_v, lab_v, valid_v, mask_v, full_v,
                    depth_v, la_v, prop_v, pos_v, out_v, *, B, S):
    wid = lax.axis_index("s") * _NUM_CORES + lax.axis_index("c")

    @pl.when(wid < B)
    def _():
        row = wid
        pltpu.sync_copy(pred_hbm.at[row], pred_v)
        pltpu.sync_copy(lab_hbm.at[row], lab_v)
        pltpu.sync_copy(valid_hbm.at[row], valid_v)
        pltpu.sync_copy(mask_hbm.at[row], mask_v)
        pltpu.sync_copy(full_hbm.at[row], full_v)
        pltpu.sync_copy(depth_hbm, depth_v)

        d = depth_v[pl.ds(0, _LANES)]
        n_chunks = S // _LANES

        def body1(i, carry):
            sl = pl.ds(i * _LANES, _LANES)
            pred = pred_v[sl]
            lab = lab_v[sl]
            valid = valid_v[sl]
            mv = mask_v[sl]
            s_glob = lax.iota(jnp.int32, _LANES) + i * _LANES
            cont = (pred != lab) & (s_glob < S - 1) & (lab != _IGNORE_INDEX)
            la = jnp.where(cont, d + 1, d)
            la = jnp.minimum(la, _MAX_ITER)
            la = jnp.where(valid == 1, la, _IGNORE_INDEX)
            la_v[sl] = la
            prop_v[sl] = jnp.where(la == _IGNORE_INDEX, 0, la)
            cs = plsc.cumsum(mv) + carry
            pos_v[sl] = jnp.clip(cs - 1, 0, S - 1)
            return jnp.max(cs)

        lax.fori_loop(0, n_chunks, body1, jnp.int32(0))

        def body2(i, carry):
            sl = pl.ds(i * _LANES, _LANES)
            pos = pos_v[sl]
            g = plsc.load_gather(prop_v, [pos])
            mv = mask_v[sl]
            fv = full_v[sl]
            out_v[sl] = jnp.maximum(fv, jnp.where(mv != 0, g, 0))
            return carry

        lax.fori_loop(0, n_chunks, body2, jnp.int32(0))

        pltpu.sync_copy(la_v, la_hbm.at[row])
        pltpu.sync_copy(out_v, full_out_hbm.at[row])


def kernel(active_logits, active_labels_shifted, iter_depth,
           current_iter_mask, active_valid_mask, full_labels):
    B, S, V = active_logits.shape
    BS = 512
    n_sblk = S // BS

    predicted = pl.pallas_call(
        _argmax_body,
        grid=(B, n_sblk),
        in_specs=[pl.BlockSpec((1, BS, V), lambda b, s: (b, s, 0))],
        out_specs=pl.BlockSpec((1, BS, 1), lambda b, s: (b * (S // BS) + s, 0, 0)),
        out_shape=jax.ShapeDtypeStruct((B * n_sblk, BS, 1), jnp.int32),
    )(active_logits)
    predicted = predicted.reshape(B, S)

    lab = active_labels_shifted.astype(jnp.int32)
    valid = active_valid_mask.astype(jnp.int32)
    maskv = current_iter_mask.astype(jnp.int32)
    full = full_labels.astype(jnp.int32)
    depth = jnp.full((_LANES,), iter_depth, dtype=jnp.int32)

    mesh = plsc.VectorSubcoreMesh(
        core_axis_name="c", subcore_axis_name="s",
        num_cores=_NUM_CORES, num_subcores=_NUM_SUBCORES)
    row_i32 = functools.partial(pltpu.VMEM, (S,), jnp.int32)
    sc_call = pl.kernel(
        functools.partial(_sc_assign_body, B=B, S=S),
        out_type=[jax.ShapeDtypeStruct((B, S), jnp.int32),
                  jax.ShapeDtypeStruct((B, S), jnp.int32)],
        mesh=mesh,
        scratch_types=[row_i32(), row_i32(), row_i32(), row_i32(), row_i32(),
                       pltpu.VMEM((_LANES,), jnp.int32),
                       row_i32(), row_i32(), row_i32(), row_i32()],
        compiler_params=pltpu.CompilerParams(needs_layout_passes=False),
    )
    la, full_new = sc_call(predicted, lab, valid, maskv, full, depth)
    return la, full_new


# speedup vs baseline: 1.8184x; 1.1666x over previous
import functools

import jax
import jax.numpy as jnp
from jax import lax
from jax.experimental import pallas as pl
from jax.experimental.pallas import tpu as pltpu
from jax.experimental.pallas import tpu_sc as plsc

_IGNORE_INDEX = -100
_MAX_ITER = 3
_LANES = 16
_NUM_CORES = 2
_NUM_SUBCORES = 16


def _argmax_body(x_ref, out_ref):
    x = x_ref[0]
    pred = jnp.argmax(x, axis=-1, keepdims=True)
    out_ref[0] = pred.astype(jnp.int32)


def _sc_assign_body(pred_hbm, lab_hbm, valid_hbm, mask_hbm, full_hbm,
                    depth_hbm, la_hbm, full_out_hbm,
                    pred_v, lab_v, valid_v, mask_v, full_v,
                    depth_v, la_v, prop_v, pos_v, out_v, *, B, S):
    wid = lax.axis_index("s") * _NUM_CORES + lax.axis_index("c")

    @pl.when(wid < B)
    def _():
        row = wid
        pltpu.sync_copy(pred_hbm.at[row], pred_v)
        pltpu.sync_copy(lab_hbm.at[row], lab_v)
        pltpu.sync_copy(valid_hbm.at[row], valid_v)
        pltpu.sync_copy(mask_hbm.at[row], mask_v)
        pltpu.sync_copy(full_hbm.at[row], full_v)
        pltpu.sync_copy(depth_hbm, depth_v)

        d = depth_v[pl.ds(0, _LANES)]
        n_chunks = S // _LANES

        def body1(i, carry):
            sl = pl.ds(i * _LANES, _LANES)
            pred = pred_v[sl]
            lab = lab_v[sl]
            valid = valid_v[sl]
            mv = mask_v[sl]
            s_glob = lax.iota(jnp.int32, _LANES) + i * _LANES
            cont = (pred != lab) & (s_glob < S - 1) & (lab != _IGNORE_INDEX)
            la = jnp.where(cont, d + 1, d)
            la = jnp.minimum(la, _MAX_ITER)
            la = jnp.where(valid == 1, la, _IGNORE_INDEX)
            la_v[sl] = la
            prop_v[sl] = jnp.where(la == _IGNORE_INDEX, 0, la)
            cs = plsc.cumsum(mv) + carry
            pos_v[sl] = jnp.clip(cs - 1, 0, S - 1)
            return jnp.max(cs)

        lax.fori_loop(0, n_chunks, body1, jnp.int32(0))

        def body2(i, carry):
            sl = pl.ds(i * _LANES, _LANES)
            pos = pos_v[sl]
            g = plsc.load_gather(prop_v, [pos])
            mv = mask_v[sl]
            fv = full_v[sl]
            out_v[sl] = jnp.maximum(fv, jnp.where(mv != 0, g, 0))
            return carry

        lax.fori_loop(0, n_chunks, body2, jnp.int32(0))

        pltpu.sync_copy(la_v, la_hbm.at[row])
        pltpu.sync_copy(out_v, full_out_hbm.at[row])


def kernel(active_logits, active_labels_shifted, iter_depth,
           current_iter_mask, active_valid_mask, full_labels):
    B, S, V = active_logits.shape
    BS = 1024
    n_sblk = S // BS

    predicted = pl.pallas_call(
        _argmax_body,
        grid=(B, n_sblk),
        in_specs=[pl.BlockSpec((1, BS, V), lambda b, s: (b, s, 0))],
        out_specs=pl.BlockSpec((1, BS, 1), lambda b, s: (b * (S // BS) + s, 0, 0)),
        out_shape=jax.ShapeDtypeStruct((B * n_sblk, BS, 1), jnp.int32),
    )(active_logits)
    predicted = predicted.reshape(B, S)

    lab = active_labels_shifted.astype(jnp.int32)
    valid = active_valid_mask.astype(jnp.int32)
    maskv = current_iter_mask.astype(jnp.int32)
    full = full_labels.astype(jnp.int32)
    depth = jnp.full((_LANES,), iter_depth, dtype=jnp.int32)

    mesh = plsc.VectorSubcoreMesh(
        core_axis_name="c", subcore_axis_name="s",
        num_cores=_NUM_CORES, num_subcores=_NUM_SUBCORES)
    row_i32 = functools.partial(pltpu.VMEM, (S,), jnp.int32)
    sc_call = pl.kernel(
        functools.partial(_sc_assign_body, B=B, S=S),
        out_type=[jax.ShapeDtypeStruct((B, S), jnp.int32),
                  jax.ShapeDtypeStruct((B, S), jnp.int32)],
        mesh=mesh,
        scratch_types=[row_i32(), row_i32(), row_i32(), row_i32(), row_i32(),
                       pltpu.VMEM((_LANES,), jnp.int32),
                       row_i32(), row_i32(), row_i32(), row_i32()],
        compiler_params=pltpu.CompilerParams(needs_layout_passes=False),
    )
    la, full_new = sc_call(predicted, lab, valid, maskv, full, depth)
    return la, full_new


# speedup vs baseline: 1.8382x; 1.0109x over previous
import functools

import jax
import jax.numpy as jnp
from jax import lax
from jax.experimental import pallas as pl
from jax.experimental.pallas import tpu as pltpu
from jax.experimental.pallas import tpu_sc as plsc

_IGNORE_INDEX = -100
_MAX_ITER = 3
_LANES = 16
_NUM_CORES = 2
_NUM_SUBCORES = 16


def _argmax_body(x_ref, out_ref):
    x = x_ref[0]
    pred = jnp.argmax(x, axis=-1, keepdims=True)
    out_ref[0] = pred.astype(jnp.int32)


def _sc_assign_body(pred_hbm, lab_hbm, valid_hbm, mask_hbm, full_hbm,
                    depth_hbm, la_hbm, full_out_hbm,
                    pred_v, lab_v, valid_v, mask_v, full_v,
                    depth_v, la_v, prop_v, pos_v, out_v, *, B, S):
    wid = lax.axis_index("s") * _NUM_CORES + lax.axis_index("c")

    @pl.when(wid < B)
    def _():
        row = wid
        pltpu.sync_copy(pred_hbm.at[row], pred_v)
        pltpu.sync_copy(lab_hbm.at[row], lab_v)
        pltpu.sync_copy(valid_hbm.at[row], valid_v)
        pltpu.sync_copy(mask_hbm.at[row], mask_v)
        pltpu.sync_copy(full_hbm.at[row], full_v)
        pltpu.sync_copy(depth_hbm, depth_v)

        d = depth_v[pl.ds(0, _LANES)]
        n_chunks = S // _LANES

        def body1(i, carry):
            sl = pl.ds(i * _LANES, _LANES)
            pred = pred_v[sl]
            lab = lab_v[sl]
            valid = valid_v[sl]
            mv = mask_v[sl]
            s_glob = lax.iota(jnp.int32, _LANES) + i * _LANES
            cont = (pred != lab) & (s_glob < S - 1) & (lab != _IGNORE_INDEX)
            la = jnp.where(cont, d + 1, d)
            la = jnp.minimum(la, _MAX_ITER)
            la = jnp.where(valid == 1, la, _IGNORE_INDEX)
            la_v[sl] = la
            prop_v[sl] = jnp.where(la == _IGNORE_INDEX, 0, la)
            cs = plsc.cumsum(mv) + carry
            pos_v[sl] = jnp.clip(cs - 1, 0, S - 1)
            return jnp.max(cs)

        lax.fori_loop(0, n_chunks, body1, jnp.int32(0))

        def body2(i, carry):
            sl = pl.ds(i * _LANES, _LANES)
            pos = pos_v[sl]
            g = plsc.load_gather(prop_v, [pos])
            mv = mask_v[sl]
            fv = full_v[sl]
            out_v[sl] = jnp.maximum(fv, jnp.where(mv != 0, g, 0))
            return carry

        lax.fori_loop(0, n_chunks, body2, jnp.int32(0))

        pltpu.sync_copy(la_v, la_hbm.at[row])
        pltpu.sync_copy(out_v, full_out_hbm.at[row])


def kernel(active_logits, active_labels_shifted, iter_depth,
           current_iter_mask, active_valid_mask, full_labels):
    B, S, V = active_logits.shape
    BS = 2048
    n_sblk = S // BS

    predicted = pl.pallas_call(
        _argmax_body,
        grid=(B, n_sblk),
        in_specs=[pl.BlockSpec((1, BS, V), lambda b, s: (b, s, 0))],
        out_specs=pl.BlockSpec((1, BS, 1), lambda b, s: (b * (S // BS) + s, 0, 0)),
        out_shape=jax.ShapeDtypeStruct((B * n_sblk, BS, 1), jnp.int32),
        compiler_params=pltpu.CompilerParams(
            vmem_limit_bytes=100 * 1024 * 1024),
    )(active_logits)
    predicted = predicted.reshape(B, S)

    lab = active_labels_shifted.astype(jnp.int32)
    valid = active_valid_mask.astype(jnp.int32)
    maskv = current_iter_mask.astype(jnp.int32)
    full = full_labels.astype(jnp.int32)
    depth = jnp.full((_LANES,), iter_depth, dtype=jnp.int32)

    mesh = plsc.VectorSubcoreMesh(
        core_axis_name="c", subcore_axis_name="s",
        num_cores=_NUM_CORES, num_subcores=_NUM_SUBCORES)
    row_i32 = functools.partial(pltpu.VMEM, (S,), jnp.int32)
    sc_call = pl.kernel(
        functools.partial(_sc_assign_body, B=B, S=S),
        out_type=[jax.ShapeDtypeStruct((B, S), jnp.int32),
                  jax.ShapeDtypeStruct((B, S), jnp.int32)],
        mesh=mesh,
        scratch_types=[row_i32(), row_i32(), row_i32(), row_i32(), row_i32(),
                       pltpu.VMEM((_LANES,), jnp.int32),
                       row_i32(), row_i32(), row_i32(), row_i32()],
        compiler_params=pltpu.CompilerParams(needs_layout_passes=False),
    )
    la, full_new = sc_call(predicted, lab, valid, maskv, full, depth)
    return la, full_new
